# baseline (speedup 1.0000x reference)
import functools

import jax
import jax.numpy as jnp
from jax import lax
from jax.experimental import pallas as pl
from jax.experimental.pallas import tpu as pltpu
from jax.experimental.pallas import tpu_sc as plsc

N_NODES = 50000
N_EDGES = 800000
HIDDEN = 32
G = 8
GH = G * HIDDEN

NODE_BLK = 10000
EBLK = 512

NW = 32
EDGES_PAD = 819200
ROWS8 = EDGES_PAD // G
EPW = EDGES_PAD // NW
GCH = 512
GU = GCH // 128
GNCH = EPW // GCH
TAB_ROWS = 50048
TLOAD = TAB_ROWS // 16
OWN = 25024
ACC_ROWS = 25088
ZROWS = ACC_ROWS // 16
SCH = 512
SU = SCH // 128
SC_EPW = EDGES_PAD // 16
SC_NCH = SC_EPW // SCH

_LEAK = 0.05
_EPS = 1e-5


def _leaky(x):
    return jnp.where(x >= 0, x, _LEAK * x)


def _dot(a, b):
    return jnp.dot(a, b, preferred_element_type=jnp.float32)


_SC_PARAMS = pltpu.CompilerParams(use_tc_tiling_on_sc=False)


@functools.cache
def _sc_mesh():
    return plsc.VectorSubcoreMesh(core_axis_name="c", subcore_axis_name="s")


def _gather_body(hnv, idx2, gsv, gdv,
                 table, isrc0, idst0, isrc1, idst1,
                 rs0, rd0, rs1, rd1,
                 gsem, isem0, isem1, wsem0, wsem1):
    s = lax.axis_index("s")
    w = s * 2 + lax.axis_index("c")

    def prefetch_idx(i, isrc, idst, isem):
        row = w * GNCH + i
        pltpu.async_copy(idx2.at[0, row], isrc, isem)
        pltpu.async_copy(idx2.at[1, row], idst, isem)

    prefetch_idx(0, isrc0, idst0, isem0)
    prefetch_idx(1, isrc1, idst1, isem1)
    tsl = pl.ds(s * TLOAD, TLOAD)
    pltpu.sync_copy(hnv.at[tsl], table.at[tsl])
    plsc.subcore_barrier()

    def phase(it, b, isrc, idst, rs, rd, isem, wsem):
        i = it * 2 + b

        @pl.when(it >= 1)
        def _():
            pltpu.make_async_copy(rs, gsv.at[pl.ds(0, GCH)], wsem).wait()
            pltpu.make_async_copy(rd, gdv.at[pl.ds(0, GCH)], wsem).wait()

        pltpu.make_async_copy(idx2.at[0, 0], isrc, isem).wait()
        pltpu.make_async_copy(idx2.at[1, 0], idst, isem).wait()

        cps = []
        for j in range(GU):
            sl = pl.ds(j * 128, 128)
            cps.append(pltpu.async_copy(table.at[isrc.at[j]], rs.at[sl],
                                        gsem))
            cps.append(pltpu.async_copy(table.at[idst.at[j]], rd.at[sl],
                                        gsem))
        for cp in cps:
            cp.wait()

        @pl.when(it < GNCH // 2 - 1)
        def _():
            prefetch_idx(i + 2, isrc, idst, isem)

        ebase = w * EPW + i * GCH
        pltpu.async_copy(rs, gsv.at[pl.ds(ebase, GCH)], wsem)
        pltpu.async_copy(rd, gdv.at[pl.ds(ebase, GCH)], wsem)

    def body(it, carry):
        phase(it, 0, isrc0, idst0, rs0, rd0, isem0, wsem0)
        phase(it, 1, isrc1, idst1, rs1, rd1, isem1, wsem1)
        return carry

    lax.fori_loop(0, GNCH // 2, body, 0)
    pltpu.make_async_copy(rs0, gsv.at[pl.ds(0, GCH)], wsem0).wait()
    pltpu.make_async_copy(rd0, gdv.at[pl.ds(0, GCH)], wsem0).wait()
    pltpu.make_async_copy(rs1, gsv.at[pl.ds(0, GCH)], wsem1).wait()
    pltpu.make_async_copy(rd1, gdv.at[pl.ds(0, GCH)], wsem1).wait()


def _sc_gather(hnp, idx2):
    out_t = jax.ShapeDtypeStruct((EDGES_PAD, HIDDEN // 2), jnp.float32)
    f = pl.kernel(
        _gather_body,
        out_type=[out_t, out_t],
        mesh=_sc_mesh(),
        scratch_types=[
            pltpu.VMEM_SHARED((TAB_ROWS, HIDDEN // 2), jnp.float32),
            pltpu.VMEM((GU, 128), jnp.int32),
            pltpu.VMEM((GU, 128), jnp.int32),
            pltpu.VMEM((GU, 128), jnp.int32),
            pltpu.VMEM((GU, 128), jnp.int32),
            pltpu.VMEM((GCH, HIDDEN // 2), jnp.float32),
            pltpu.VMEM((GCH, HIDDEN // 2), jnp.float32),
            pltpu.VMEM((GCH, HIDDEN // 2), jnp.float32),
            pltpu.VMEM((GCH, HIDDEN // 2), jnp.float32),
            pltpu.SemaphoreType.DMA,
            pltpu.SemaphoreType.DMA,
            pltpu.SemaphoreType.DMA,
            pltpu.SemaphoreType.DMA,
            pltpu.SemaphoreType.DMA,
        ],
        compiler_params=_SC_PARAMS,
    )
    return f(hnp, idx2)


def _scatter_body(he, idx2, zeros, out,
                  idx, rows, acc, ssem, isem, rsem):
    c = lax.axis_index("c")
    s = lax.axis_index("s")
    base = c * OWN

    def prefetch(i, b):
        pltpu.async_copy(idx2.at[0, s * SC_NCH + i], idx[b], isem[b])
        pltpu.async_copy(he.at[pl.ds(s * SC_EPW + i * SCH, SCH)], rows[b],
                         rsem[b])

    def drain_scatter(b):
        for j in range(SU):
            pltpu.make_async_copy(
                rows[b].at[pl.ds(j * 128, 128)],
                acc.at[idx[b].at[j]], ssem[b]).wait()

    prefetch(0, 0)
    prefetch(1, 1)
    pltpu.sync_copy(zeros, acc.at[pl.ds(s * ZROWS, ZROWS)])
    plsc.subcore_barrier()

    def phase(it, b):
        i = it * 4 + b
        pltpu.make_async_copy(idx2.at[0, 0], idx[b], isem[b]).wait()
        pltpu.make_async_copy(he.at[pl.ds(0, SCH)], rows[b], rsem[b]).wait()
        for j in range(SU):
            for k in range(8):
                sl = pl.ds(k * 16, 16)
                v = idx[b][j, sl] - base
                oob = (v < 0) | (v >= OWN)
                idx[b][j, sl] = jnp.where(oob, OWN, v)
        for j in range(SU):
            pltpu.async_copy(
                rows[b].at[pl.ds(j * 128, 128)], acc.at[idx[b].at[j]],
                ssem[b], add=True)
        b2 = (b + 2) % 4
        if b >= 2:
            drain_scatter(b2)

            @pl.when(it < SC_NCH // 4 - 1)
            def _():
                prefetch(i + 2, b2)
        else:
            @pl.when(it >= 1)
            def _():
                drain_scatter(b2)

            prefetch(i + 2, b2)

    def body(it, carry):
        for b in range(4):
            phase(it, b)
        return carry

    lax.fori_loop(0, SC_NCH // 4, body, 0)
    drain_scatter(2)
    drain_scatter(3)
    plsc.subcore_barrier()

    @pl.when(c == 0)
    def _():
        rpt = OWN // 16
        pltpu.sync_copy(acc.at[pl.ds(s * rpt, rpt)],
                        out.at[pl.ds(s * rpt, rpt)])

    @pl.when(c == 1)
    def _():
        rpt = (N_NODES - OWN) // 16
        pltpu.sync_copy(acc.at[pl.ds(s * rpt, rpt)],
                        out.at[pl.ds(OWN + s * rpt, rpt)])


def _sc_scatter(he32, idx2, zeros):
    f = pl.kernel(
        _scatter_body,
        out_type=jax.ShapeDtypeStruct((N_NODES, HIDDEN), jnp.float32),
        mesh=_sc_mesh(),
        scratch_types=[
            [pltpu.VMEM((SU, 128), jnp.int32) for _ in range(4)],
            [pltpu.VMEM((SCH, HIDDEN), jnp.float32) for _ in range(4)],
            pltpu.VMEM_SHARED((ACC_ROWS, HIDDEN), jnp.float32),
            [pltpu.SemaphoreType.DMA for _ in range(4)],
            [pltpu.SemaphoreType.DMA for _ in range(4)],
            [pltpu.SemaphoreType.DMA for _ in range(4)],
        ],
        compiler_params=_SC_PARAMS,
    )
    return f(he32, idx2, zeros)


def _kron8(w):
    return jnp.kron(jnp.eye(G, dtype=jnp.float32), w)


def _tile8(v):
    return jnp.tile(v.reshape(1, -1), (1, G))


def _group_ln(x, g8, bb8, mdown, mup):
    s = _dot(x, mdown)
    mu = _dot(s, mup)
    xc = x - mu
    v = _dot(xc * xc, mdown)
    var = _dot(v, mup)
    return xc * lax.rsqrt(var + _EPS) * g8 + bb8


ROW_REAL = N_EDGES // G


def _pad_mask(x):
    rid = pl.program_id(0) * EBLK + lax.broadcasted_iota(
        jnp.int32, (EBLK, 1), 0)
    return jnp.where(rid < ROW_REAL, x, 0.0)


def _unpack_bf16(ref):
    wi = lax.bitcast_convert_type(ref[...], jnp.int32)
    ev = lax.bitcast_convert_type(
        lax.shift_left(wi, 16), jnp.float32).astype(jnp.bfloat16)
    od = lax.bitcast_convert_type(
        wi & jnp.int32(-65536), jnp.float32).astype(jnp.bfloat16)
    return ev, od


def _edge_enc_body(x_ref, w1_ref, b1_ref, w2_ref, b2_ref, g_ref, bb_ref,
                   md_ref, mu_ref, o_ref):
    xb = x_ref[...].astype(jnp.bfloat16)
    h = _leaky(_dot(xb, w1_ref[...]) + b1_ref[...])
    h2 = _leaky(_dot(h.astype(jnp.bfloat16), w2_ref[...]) + b2_ref[...])
    o_ref[...] = _pad_mask(_group_ln(h2, g_ref[...], bb_ref[...],
                                     md_ref[...], mu_ref[...]))


def _edge_enc(e8, mlp):
    (w1, b1), (w2, b2) = mlp["lin"]
    g, bb = mlp["ln"]
    w1k = _kron8(w1).astype(jnp.bfloat16)
    w2k = _kron8(w2).astype(jnp.bfloat16)
    mdown = _kron8(jnp.full((HIDDEN, 1), 1.0 / HIDDEN))
    mup = _kron8(jnp.ones((1, HIDDEN), jnp.float32))
    grid = ROWS8 // EBLK
    full = lambda shape: pl.BlockSpec(shape, lambda i: (0, 0))
    return pl.pallas_call(
        _edge_enc_body,
        grid=(grid,),
        in_specs=[
            pl.BlockSpec((EBLK, G * 4), lambda i: (i, 0)),
            full(w1k.shape), full((1, GH)),
            full(w2k.shape), full((1, GH)),
            full((1, GH)), full((1, GH)),
            full(mdown.shape), full(mup.shape),
        ],
        out_specs=pl.BlockSpec((EBLK, GH), lambda i: (i, 0)),
        out_shape=jax.ShapeDtypeStruct((ROWS8, GH), jnp.float32),
    )(e8, w1k, _tile8(b1), w2k, _tile8(b2), _tile8(g), _tile8(bb),
      mdown, mup)


def _edge_msg_body(he_ref, gs_ref, gd_ref, w1e_ref, w1se_ref, w1so_ref,
                   w1de_ref, w1do_ref, b1_ref,
                   w2_ref, b2_ref, g_ref, bb_ref, md_ref, mu_ref, o_ref):
    he = he_ref[...]
    gse, gso = _unpack_bf16(gs_ref)
    gde, gdo = _unpack_bf16(gd_ref)
    x1 = (_dot(he.astype(jnp.bfloat16), w1e_ref[...])
          + _dot(gse, w1se_ref[...]) + _dot(gso, w1so_ref[...])
          + _dot(gde, w1de_ref[...]) + _dot(gdo, w1do_ref[...])
          + b1_ref[...])
    h = _leaky(x1)
    h2 = _leaky(_dot(h.astype(jnp.bfloat16), w2_ref[...]) + b2_ref[...])
    y = _group_ln(h2, g_ref[...], bb_ref[...], md_ref[...], mu_ref[...])
    o_ref[...] = _pad_mask(he + y)


def _edge_msg(he8, gsp, gdp, mlp):
    (w1, b1), (w2, b2) = mlp["lin"]
    g, bb = mlp["ln"]
    w1e = _kron8(w1[:HIDDEN]).astype(jnp.bfloat16)
    w1s = _kron8(w1[HIDDEN:2 * HIDDEN])
    w1d = _kron8(w1[2 * HIDDEN:])
    w1se = w1s[0::2].astype(jnp.bfloat16)
    w1so = w1s[1::2].astype(jnp.bfloat16)
    w1de = w1d[0::2].astype(jnp.bfloat16)
    w1do = w1d[1::2].astype(jnp.bfloat16)
    w2k = _kron8(w2).astype(jnp.bfloat16)
    mdown = _kron8(jnp.full((HIDDEN, 1), 1.0 / HIDDEN))
    mup = _kron8(jnp.ones((1, HIDDEN), jnp.float32))
    grid = ROWS8 // EBLK
    row = pl.BlockSpec((EBLK, GH), lambda i: (i, 0))
    rowp = pl.BlockSpec((EBLK, 128), lambda i: (i, 0))
    full = lambda shape: pl.BlockSpec(shape, lambda i: (0, 0))
    w = pl.BlockSpec((GH, GH), lambda i: (0, 0))
    wh = pl.BlockSpec((128, GH), lambda i: (0, 0))
    return pl.pallas_call(
        _edge_msg_body,
        grid=(grid,),
        in_specs=[row, rowp, rowp, w, wh, wh, wh, wh, full((1, GH)), w,
                  full((1, GH)), full((1, GH)), full((1, GH)),
                  full(mdown.shape), full(mup.shape)],
        out_specs=row,
        out_shape=jax.ShapeDtypeStruct((ROWS8, GH), jnp.float32),
    )(he8, gsp, gdp, w1e, w1se, w1so, w1de, w1do, _tile8(b1), w2k,
      _tile8(b2), _tile8(g), _tile8(bb), mdown, mup)


def _ln32(x, g, b):
    mu = jnp.mean(x, axis=-1, keepdims=True)
    var = jnp.mean((x - mu) ** 2, axis=-1, keepdims=True)
    return (x - mu) * lax.rsqrt(var + _EPS) * g + b


def _node_mlp2_body(x_ref, w1_ref, b1_ref, w2_ref, b2_ref, g_ref, bb_ref,
                    o_ref, ob_ref):
    h = _leaky(_dot(x_ref[...], w1_ref[...]) + b1_ref[...])
    h = _leaky(_dot(h, w2_ref[...]) + b2_ref[...])
    h = _ln32(h, g_ref[...], bb_ref[...])
    o_ref[...] = h
    ob_ref[...] = h.astype(jnp.bfloat16)


def _node_enc(x, mlp):
    (w1, b1), (w2, b2) = mlp["lin"]
    g, bb = mlp["ln"]
    n, fin = x.shape
    blk = NODE_BLK
    grid = n // blk
    full = lambda shape: pl.BlockSpec(shape, lambda i: (0, 0))
    row = pl.BlockSpec((blk, HIDDEN), lambda i: (i, 0))
    return pl.pallas_call(
        _node_mlp2_body,
        grid=(grid,),
        in_specs=[
            pl.BlockSpec((blk, fin), lambda i: (i, 0)),
            full(w1.shape), full((1, HIDDEN)),
            full(w2.shape), full((1, HIDDEN)),
            full((1, HIDDEN)), full((1, HIDDEN)),
        ],
        out_specs=[row, row],
        out_shape=[jax.ShapeDtypeStruct((N_NODES, HIDDEN), jnp.float32),
                   jax.ShapeDtypeStruct((TAB_ROWS, HIDDEN), jnp.bfloat16)],
    )(x, w1, b1.reshape(1, -1), w2, b2.reshape(1, -1),
      g.reshape(1, -1), bb.reshape(1, -1))


def _node_msg(hn, aggr, mlp):
    (w1, b1), (w2, b2) = mlp["lin"]
    g, bb = mlp["ln"]
    w1n, w1a = w1[:HIDDEN], w1[HIDDEN:]
    grid = N_NODES // NODE_BLK
    row = pl.BlockSpec((NODE_BLK, HIDDEN), lambda i: (i, 0))
    full = lambda shape: pl.BlockSpec(shape, lambda i: (0, 0))
    w = pl.BlockSpec((HIDDEN, HIDDEN), lambda i: (0, 0))

    def body(hn_ref, a_ref, w1n_ref, w1a_ref, b1_ref,
             w2_ref, b2_ref, g_ref, bb_ref, o_ref, ob_ref):
        hn = hn_ref[...]
        x1 = (_dot(hn, w1n_ref[...]) + _dot(a_ref[...], w1a_ref[...])
              + b1_ref[...])
        h = _leaky(x1)
        h = _leaky(_dot(h, w2_ref[...]) + b2_ref[...])
        o = hn + _ln32(h, g_ref[...], bb_ref[...])
        o_ref[...] = o
        ob_ref[...] = o.astype(jnp.bfloat16)

    return pl.pallas_call(
        body,
        grid=(grid,),
        in_specs=[row, row, w, w, full((1, HIDDEN)), w,
                  full((1, HIDDEN)), full((1, HIDDEN)), full((1, HIDDEN))],
        out_specs=[row, row],
        out_shape=[jax.ShapeDtypeStruct((N_NODES, HIDDEN), jnp.float32),
                   jax.ShapeDtypeStruct((TAB_ROWS, HIDDEN), jnp.bfloat16)],
    )(hn, aggr, w1n, w1a, b1.reshape(1, -1), w2, b2.reshape(1, -1),
      g.reshape(1, -1), bb.reshape(1, -1))


def _decoder_body(x_ref, wa_ref, ba_ref, wb_ref, bb_ref, o_ref):
    h = _leaky(_dot(x_ref[...], wa_ref[...]) + ba_ref[...])
    o_ref[...] = _dot(h, wb_ref[...]) + bb_ref[...]


def _decoder(hn, dec):
    (wa, ba), (wb, bb) = dec["lin"]
    out_w = wb.shape[1]
    grid = N_NODES // NODE_BLK
    full = lambda shape: pl.BlockSpec(shape, lambda i: (0, 0))
    return pl.pallas_call(
        _decoder_body,
        grid=(grid,),
        in_specs=[pl.BlockSpec((NODE_BLK, HIDDEN), lambda i: (i, 0)),
                  full(wa.shape), full((1, HIDDEN)),
                  full(wb.shape), full((1, out_w))],
        out_specs=pl.BlockSpec((NODE_BLK, out_w), lambda i: (i, 0)),
        out_shape=jax.ShapeDtypeStruct((N_NODES, out_w), jnp.float32),
    )(hn, wa, ba.reshape(1, -1), wb, bb.reshape(1, -1))


def _pack_table(hnb):
    return lax.bitcast_convert_type(
        hnb.reshape(TAB_ROWS, HIDDEN // 2, 2), jnp.float32)


def kernel(nodes, edges, edge_idx, coarse_edges, coarse_edge_idx, aggr_weights,
           aggr_edge_idx, int_weights, int_receivers, int_senders, params):
    del coarse_edges, coarse_edge_idx, aggr_weights, aggr_edge_idx
    del int_weights, int_receivers, int_senders

    npad = EDGES_PAD - N_EDGES
    idx2 = jnp.stack(
        [jnp.pad(edge_idx[:, 0].astype(jnp.int32), (0, npad)),
         jnp.pad(edge_idx[:, 1].astype(jnp.int32), (0, npad))]).reshape(
             2, NW * GNCH, GU, 128)
    zeros = jnp.zeros((ZROWS, HIDDEN), jnp.float32)
    e8 = jnp.pad(edges, ((0, npad), (0, 0))).reshape(ROWS8, G * 4)

    hn, hnb = _node_enc(nodes, params["node_enc"])
    he8 = _edge_enc(e8, params["edge_enc"])

    for i in range(4):
        gs16, gd16 = _sc_gather(_pack_table(hnb), idx2)
        he8 = _edge_msg(he8, gs16.reshape(ROWS8, 128),
                        gd16.reshape(ROWS8, 128), params["edge_msg"][i])
        aggr = _sc_scatter(he8.reshape(EDGES_PAD, HIDDEN), idx2, zeros)
        hn, hnb = _node_msg(hn, aggr, params["node_msg"][i])

    return _decoder(hn, params["decoder"])


# speedup vs baseline: 1.0361x; 1.0361x over previous
import functools

import jax
import jax.numpy as jnp
from jax import lax
from jax.experimental import pallas as pl
from jax.experimental.pallas import tpu as pltpu
from jax.experimental.pallas import tpu_sc as plsc

N_NODES = 50000
N_EDGES = 800000
HIDDEN = 32
G = 8
GH = G * HIDDEN

NODE_BLK = 10000
EBLK = 1024

NW = 32
EDGES_PAD = 819200
ROWS8 = EDGES_PAD // G
EPW = EDGES_PAD // NW
GCH = 512
GU = GCH // 128
GNCH = EPW // GCH
TAB_ROWS = 50048
TLOAD = TAB_ROWS // 16
OWN = 25024
ACC_ROWS = 25088
ZROWS = ACC_ROWS // 16
SCH = 512
SU = SCH // 128
SC_EPW = EDGES_PAD // 16
SC_NCH = SC_EPW // SCH

_LEAK = 0.05
_EPS = 1e-5


def _leaky(x):
    return jnp.where(x >= 0, x, _LEAK * x)


def _dot(a, b):
    return jnp.dot(a, b, preferred_element_type=jnp.float32)


_SC_PARAMS = pltpu.CompilerParams(use_tc_tiling_on_sc=False)


@functools.cache
def _sc_mesh():
    return plsc.VectorSubcoreMesh(core_axis_name="c", subcore_axis_name="s")


def _gather_body(hnv, idx2, gsv, gdv,
                 table, isrc0, idst0, isrc1, idst1,
                 rs0, rd0, rs1, rd1,
                 gsem, isem0, isem1, wsem0, wsem1):
    s = lax.axis_index("s")
    w = s * 2 + lax.axis_index("c")

    def prefetch_idx(i, isrc, idst, isem):
        row = w * GNCH + i
        pltpu.async_copy(idx2.at[0, row], isrc, isem)
        pltpu.async_copy(idx2.at[1, row], idst, isem)

    prefetch_idx(0, isrc0, idst0, isem0)
    prefetch_idx(1, isrc1, idst1, isem1)
    tsl = pl.ds(s * TLOAD, TLOAD)
    pltpu.sync_copy(hnv.at[tsl], table.at[tsl])
    plsc.subcore_barrier()

    def phase(it, b, isrc, idst, rs, rd, isem, wsem):
        i = it * 2 + b

        @pl.when(it >= 1)
        def _():
            pltpu.make_async_copy(rs, gsv.at[pl.ds(0, GCH)], wsem).wait()
            pltpu.make_async_copy(rd, gdv.at[pl.ds(0, GCH)], wsem).wait()

        pltpu.make_async_copy(idx2.at[0, 0], isrc, isem).wait()
        pltpu.make_async_copy(idx2.at[1, 0], idst, isem).wait()

        cps = []
        for j in range(GU):
            sl = pl.ds(j * 128, 128)
            cps.append(pltpu.async_copy(table.at[isrc.at[j]], rs.at[sl],
                                        gsem))
            cps.append(pltpu.async_copy(table.at[idst.at[j]], rd.at[sl],
                                        gsem))
        for cp in cps:
            cp.wait()

        @pl.when(it < GNCH // 2 - 1)
        def _():
            prefetch_idx(i + 2, isrc, idst, isem)

        ebase = w * EPW + i * GCH
        pltpu.async_copy(rs, gsv.at[pl.ds(ebase, GCH)], wsem)
        pltpu.async_copy(rd, gdv.at[pl.ds(ebase, GCH)], wsem)

    def body(it, carry):
        phase(it, 0, isrc0, idst0, rs0, rd0, isem0, wsem0)
        phase(it, 1, isrc1, idst1, rs1, rd1, isem1, wsem1)
        return carry

    lax.fori_loop(0, GNCH // 2, body, 0)
    pltpu.make_async_copy(rs0, gsv.at[pl.ds(0, GCH)], wsem0).wait()
    pltpu.make_async_copy(rd0, gdv.at[pl.ds(0, GCH)], wsem0).wait()
    pltpu.make_async_copy(rs1, gsv.at[pl.ds(0, GCH)], wsem1).wait()
    pltpu.make_async_copy(rd1, gdv.at[pl.ds(0, GCH)], wsem1).wait()


def _sc_gather(hnp, idx2):
    out_t = jax.ShapeDtypeStruct((EDGES_PAD, HIDDEN // 2), jnp.float32)
    f = pl.kernel(
        _gather_body,
        out_type=[out_t, out_t],
        mesh=_sc_mesh(),
        scratch_types=[
            pltpu.VMEM_SHARED((TAB_ROWS, HIDDEN // 2), jnp.float32),
            pltpu.VMEM((GU, 128), jnp.int32),
            pltpu.VMEM((GU, 128), jnp.int32),
            pltpu.VMEM((GU, 128), jnp.int32),
            pltpu.VMEM((GU, 128), jnp.int32),
            pltpu.VMEM((GCH, HIDDEN // 2), jnp.float32),
            pltpu.VMEM((GCH, HIDDEN // 2), jnp.float32),
            pltpu.VMEM((GCH, HIDDEN // 2), jnp.float32),
            pltpu.VMEM((GCH, HIDDEN // 2), jnp.float32),
            pltpu.SemaphoreType.DMA,
            pltpu.SemaphoreType.DMA,
            pltpu.SemaphoreType.DMA,
            pltpu.SemaphoreType.DMA,
            pltpu.SemaphoreType.DMA,
        ],
        compiler_params=_SC_PARAMS,
    )
    return f(hnp, idx2)


def _scatter_body(he, idx2, zeros, out,
                  idx, rows, acc, ssem, isem, rsem):
    c = lax.axis_index("c")
    s = lax.axis_index("s")
    base = c * OWN

    def prefetch(i, b):
        pltpu.async_copy(idx2.at[0, s * SC_NCH + i], idx[b], isem[b])
        pltpu.async_copy(he.at[pl.ds(s * SC_EPW + i * SCH, SCH)], rows[b],
                         rsem[b])

    def drain_scatter(b):
        for j in range(SU):
            pltpu.make_async_copy(
                rows[b].at[pl.ds(j * 128, 128)],
                acc.at[idx[b].at[j]], ssem[b]).wait()

    prefetch(0, 0)
    prefetch(1, 1)
    pltpu.sync_copy(zeros, acc.at[pl.ds(s * ZROWS, ZROWS)])
    plsc.subcore_barrier()

    def phase(it, b):
        i = it * 4 + b
        pltpu.make_async_copy(idx2.at[0, 0], idx[b], isem[b]).wait()
        pltpu.make_async_copy(he.at[pl.ds(0, SCH)], rows[b], rsem[b]).wait()
        for j in range(SU):
            for k in range(8):
                sl = pl.ds(k * 16, 16)
                v = idx[b][j, sl] - base
                oob = (v < 0) | (v >= OWN)
                idx[b][j, sl] = jnp.where(oob, OWN, v)
        for j in range(SU):
            pltpu.async_copy(
                rows[b].at[pl.ds(j * 128, 128)], acc.at[idx[b].at[j]],
                ssem[b], add=True)
        b2 = (b + 2) % 4
        if b >= 2:
            drain_scatter(b2)

            @pl.when(it < SC_NCH // 4 - 1)
            def _():
                prefetch(i + 2, b2)
        else:
            @pl.when(it >= 1)
            def _():
                drain_scatter(b2)

            prefetch(i + 2, b2)

    def body(it, carry):
        for b in range(4):
            phase(it, b)
        return carry

    lax.fori_loop(0, SC_NCH // 4, body, 0)
    drain_scatter(2)
    drain_scatter(3)
    plsc.subcore_barrier()

    @pl.when(c == 0)
    def _():
        rpt = OWN // 16
        pltpu.sync_copy(acc.at[pl.ds(s * rpt, rpt)],
                        out.at[pl.ds(s * rpt, rpt)])

    @pl.when(c == 1)
    def _():
        rpt = (N_NODES - OWN) // 16
        pltpu.sync_copy(acc.at[pl.ds(s * rpt, rpt)],
                        out.at[pl.ds(OWN + s * rpt, rpt)])


def _sc_scatter(he32, idx2, zeros):
    f = pl.kernel(
        _scatter_body,
        out_type=jax.ShapeDtypeStruct((N_NODES, HIDDEN), jnp.float32),
        mesh=_sc_mesh(),
        scratch_types=[
            [pltpu.VMEM((SU, 128), jnp.int32) for _ in range(4)],
            [pltpu.VMEM((SCH, HIDDEN), jnp.float32) for _ in range(4)],
            pltpu.VMEM_SHARED((ACC_ROWS, HIDDEN), jnp.float32),
            [pltpu.SemaphoreType.DMA for _ in range(4)],
            [pltpu.SemaphoreType.DMA for _ in range(4)],
            [pltpu.SemaphoreType.DMA for _ in range(4)],
        ],
        compiler_params=_SC_PARAMS,
    )
    return f(he32, idx2, zeros)


def _kron8(w):
    return jnp.kron(jnp.eye(G, dtype=jnp.float32), w)


def _tile8(v):
    return jnp.tile(v.reshape(1, -1), (1, G))


def _group_ln(x, g8, bb8, mdown, mup):
    s = _dot(x, mdown)
    mu = _dot(s, mup)
    xc = x - mu
    v = _dot(xc * xc, mdown)
    var = _dot(v, mup)
    return xc * lax.rsqrt(var + _EPS) * g8 + bb8


ROW_REAL = N_EDGES // G


def _pad_mask(x):
    rid = pl.program_id(0) * EBLK + lax.broadcasted_iota(
        jnp.int32, (EBLK, 1), 0)
    return jnp.where(rid < ROW_REAL, x, 0.0)


def _unpack_bf16(ref):
    wi = lax.bitcast_convert_type(ref[...], jnp.int32)
    ev = lax.bitcast_convert_type(
        lax.shift_left(wi, 16), jnp.float32).astype(jnp.bfloat16)
    od = lax.bitcast_convert_type(
        wi & jnp.int32(-65536), jnp.float32).astype(jnp.bfloat16)
    return ev, od


def _edge_enc_body(x_ref, w1_ref, b1_ref, w2_ref, b2_ref, g_ref, bb_ref,
                   md_ref, mu_ref, o_ref):
    xb = x_ref[...].astype(jnp.bfloat16)
    h = _leaky(_dot(xb, w1_ref[...]) + b1_ref[...])
    h2 = _leaky(_dot(h.astype(jnp.bfloat16), w2_ref[...]) + b2_ref[...])
    o_ref[...] = _pad_mask(_group_ln(h2, g_ref[...], bb_ref[...],
                                     md_ref[...], mu_ref[...]))


def _edge_enc(e8, mlp):
    (w1, b1), (w2, b2) = mlp["lin"]
    g, bb = mlp["ln"]
    w1k = _kron8(w1).astype(jnp.bfloat16)
    w2k = _kron8(w2).astype(jnp.bfloat16)
    mdown = _kron8(jnp.full((HIDDEN, 1), 1.0 / HIDDEN))
    mup = _kron8(jnp.ones((1, HIDDEN), jnp.float32))
    grid = ROWS8 // EBLK
    full = lambda shape: pl.BlockSpec(shape, lambda i: (0, 0))
    return pl.pallas_call(
        _edge_enc_body,
        grid=(grid,),
        in_specs=[
            pl.BlockSpec((EBLK, G * 4), lambda i: (i, 0)),
            full(w1k.shape), full((1, GH)),
            full(w2k.shape), full((1, GH)),
            full((1, GH)), full((1, GH)),
            full(mdown.shape), full(mup.shape),
        ],
        out_specs=pl.BlockSpec((EBLK, GH), lambda i: (i, 0)),
        out_shape=jax.ShapeDtypeStruct((ROWS8, GH), jnp.float32),
    )(e8, w1k, _tile8(b1), w2k, _tile8(b2), _tile8(g), _tile8(bb),
      mdown, mup)


def _edge_msg_body(he_ref, gs_ref, gd_ref, w1e_ref, w1se_ref, w1so_ref,
                   w1de_ref, w1do_ref, b1_ref,
                   w2_ref, b2_ref, g_ref, bb_ref, md_ref, mu_ref, o_ref):
    he = he_ref[...]
    gse, gso = _unpack_bf16(gs_ref)
    gde, gdo = _unpack_bf16(gd_ref)
    x1 = (_dot(he.astype(jnp.bfloat16), w1e_ref[...])
          + _dot(gse, w1se_ref[...]) + _dot(gso, w1so_ref[...])
          + _dot(gde, w1de_ref[...]) + _dot(gdo, w1do_ref[...])
          + b1_ref[...])
    h = _leaky(x1)
    h2 = _leaky(_dot(h.astype(jnp.bfloat16), w2_ref[...]) + b2_ref[...])
    y = _group_ln(h2, g_ref[...], bb_ref[...], md_ref[...], mu_ref[...])
    o_ref[...] = _pad_mask(he + y)


def _edge_msg(he8, gsp, gdp, mlp):
    (w1, b1), (w2, b2) = mlp["lin"]
    g, bb = mlp["ln"]
    w1e = _kron8(w1[:HIDDEN]).astype(jnp.bfloat16)
    w1s = _kron8(w1[HIDDEN:2 * HIDDEN])
    w1d = _kron8(w1[2 * HIDDEN:])
    w1se = w1s[0::2].astype(jnp.bfloat16)
    w1so = w1s[1::2].astype(jnp.bfloat16)
    w1de = w1d[0::2].astype(jnp.bfloat16)
    w1do = w1d[1::2].astype(jnp.bfloat16)
    w2k = _kron8(w2).astype(jnp.bfloat16)
    mdown = _kron8(jnp.full((HIDDEN, 1), 1.0 / HIDDEN))
    mup = _kron8(jnp.ones((1, HIDDEN), jnp.float32))
    grid = ROWS8 // EBLK
    row = pl.BlockSpec((EBLK, GH), lambda i: (i, 0))
    rowp = pl.BlockSpec((EBLK, 128), lambda i: (i, 0))
    full = lambda shape: pl.BlockSpec(shape, lambda i: (0, 0))
    w = pl.BlockSpec((GH, GH), lambda i: (0, 0))
    wh = pl.BlockSpec((128, GH), lambda i: (0, 0))
    return pl.pallas_call(
        _edge_msg_body,
        grid=(grid,),
        in_specs=[row, rowp, rowp, w, wh, wh, wh, wh, full((1, GH)), w,
                  full((1, GH)), full((1, GH)), full((1, GH)),
                  full(mdown.shape), full(mup.shape)],
        out_specs=row,
        out_shape=jax.ShapeDtypeStruct((ROWS8, GH), jnp.float32),
    )(he8, gsp, gdp, w1e, w1se, w1so, w1de, w1do, _tile8(b1), w2k,
      _tile8(b2), _tile8(g), _tile8(bb), mdown, mup)


def _ln32(x, g, b):
    mu = jnp.mean(x, axis=-1, keepdims=True)
    var = jnp.mean((x - mu) ** 2, axis=-1, keepdims=True)
    return (x - mu) * lax.rsqrt(var + _EPS) * g + b


def _node_mlp2_body(x_ref, w1_ref, b1_ref, w2_ref, b2_ref, g_ref, bb_ref,
                    o_ref, ob_ref):
    h = _leaky(_dot(x_ref[...], w1_ref[...]) + b1_ref[...])
    h = _leaky(_dot(h, w2_ref[...]) + b2_ref[...])
    h = _ln32(h, g_ref[...], bb_ref[...])
    o_ref[...] = h
    ob_ref[...] = h.astype(jnp.bfloat16)


def _node_enc(x, mlp):
    (w1, b1), (w2, b2) = mlp["lin"]
    g, bb = mlp["ln"]
    n, fin = x.shape
    blk = NODE_BLK
    grid = n // blk
    full = lambda shape: pl.BlockSpec(shape, lambda i: (0, 0))
    row = pl.BlockSpec((blk, HIDDEN), lambda i: (i, 0))
    return pl.pallas_call(
        _node_mlp2_body,
        grid=(grid,),
        in_specs=[
            pl.BlockSpec((blk, fin), lambda i: (i, 0)),
            full(w1.shape), full((1, HIDDEN)),
            full(w2.shape), full((1, HIDDEN)),
            full((1, HIDDEN)), full((1, HIDDEN)),
        ],
        out_specs=[row, row],
        out_shape=[jax.ShapeDtypeStruct((N_NODES, HIDDEN), jnp.float32),
                   jax.ShapeDtypeStruct((TAB_ROWS, HIDDEN), jnp.bfloat16)],
    )(x, w1, b1.reshape(1, -1), w2, b2.reshape(1, -1),
      g.reshape(1, -1), bb.reshape(1, -1))


def _node_msg(hn, aggr, mlp):
    (w1, b1), (w2, b2) = mlp["lin"]
    g, bb = mlp["ln"]
    w1n, w1a = w1[:HIDDEN], w1[HIDDEN:]
    grid = N_NODES // NODE_BLK
    row = pl.BlockSpec((NODE_BLK, HIDDEN), lambda i: (i, 0))
    full = lambda shape: pl.BlockSpec(shape, lambda i: (0, 0))
    w = pl.BlockSpec((HIDDEN, HIDDEN), lambda i: (0, 0))

    def body(hn_ref, a_ref, w1n_ref, w1a_ref, b1_ref,
             w2_ref, b2_ref, g_ref, bb_ref, o_ref, ob_ref):
        hn = hn_ref[...]
        x1 = (_dot(hn, w1n_ref[...]) + _dot(a_ref[...], w1a_ref[...])
              + b1_ref[...])
        h = _leaky(x1)
        h = _leaky(_dot(h, w2_ref[...]) + b2_ref[...])
        o = hn + _ln32(h, g_ref[...], bb_ref[...])
        o_ref[...] = o
        ob_ref[...] = o.astype(jnp.bfloat16)

    return pl.pallas_call(
        body,
        grid=(grid,),
        in_specs=[row, row, w, w, full((1, HIDDEN)), w,
                  full((1, HIDDEN)), full((1, HIDDEN)), full((1, HIDDEN))],
        out_specs=[row, row],
        out_shape=[jax.ShapeDtypeStruct((N_NODES, HIDDEN), jnp.float32),
                   jax.ShapeDtypeStruct((TAB_ROWS, HIDDEN), jnp.bfloat16)],
    )(hn, aggr, w1n, w1a, b1.reshape(1, -1), w2, b2.reshape(1, -1),
      g.reshape(1, -1), bb.reshape(1, -1))


def _decoder_body(x_ref, wa_ref, ba_ref, wb_ref, bb_ref, o_ref):
    h = _leaky(_dot(x_ref[...], wa_ref[...]) + ba_ref[...])
    o_ref[...] = _dot(h, wb_ref[...]) + bb_ref[...]


def _decoder(hn, dec):
    (wa, ba), (wb, bb) = dec["lin"]
    out_w = wb.shape[1]
    grid = N_NODES // NODE_BLK
    full = lambda shape: pl.BlockSpec(shape, lambda i: (0, 0))
    return pl.pallas_call(
        _decoder_body,
        grid=(grid,),
        in_specs=[pl.BlockSpec((NODE_BLK, HIDDEN), lambda i: (i, 0)),
                  full(wa.shape), full((1, HIDDEN)),
                  full(wb.shape), full((1, out_w))],
        out_specs=pl.BlockSpec((NODE_BLK, out_w), lambda i: (i, 0)),
        out_shape=jax.ShapeDtypeStruct((N_NODES, out_w), jnp.float32),
    )(hn, wa, ba.reshape(1, -1), wb, bb.reshape(1, -1))


def _pack_table(hnb):
    return lax.bitcast_convert_type(
        hnb.reshape(TAB_ROWS, HIDDEN // 2, 2), jnp.float32)


def kernel(nodes, edges, edge_idx, coarse_edges, coarse_edge_idx, aggr_weights,
           aggr_edge_idx, int_weights, int_receivers, int_senders, params):
    del coarse_edges, coarse_edge_idx, aggr_weights, aggr_edge_idx
    del int_weights, int_receivers, int_senders

    npad = EDGES_PAD - N_EDGES
    idx2 = jnp.stack(
        [jnp.pad(edge_idx[:, 0].astype(jnp.int32), (0, npad)),
         jnp.pad(edge_idx[:, 1].astype(jnp.int32), (0, npad))]).reshape(
             2, NW * GNCH, GU, 128)
    zeros = jnp.zeros((ZROWS, HIDDEN), jnp.float32)
    e8 = jnp.pad(edges, ((0, npad), (0, 0))).reshape(ROWS8, G * 4)

    hn, hnb = _node_enc(nodes, params["node_enc"])
    he8 = _edge_enc(e8, params["edge_enc"])

    for i in range(4):
        gs16, gd16 = _sc_gather(_pack_table(hnb), idx2)
        he8 = _edge_msg(he8, gs16.reshape(ROWS8, 128),
                        gd16.reshape(ROWS8, 128), params["edge_msg"][i])
        aggr = _sc_scatter(he8.reshape(EDGES_PAD, HIDDEN), idx2, zeros)
        hn, hnb = _node_msg(hn, aggr, params["node_msg"][i])

    return _decoder(hn, params["decoder"])
